# Initial kernel scaffold; baseline (speedup 1.0000x reference)
import jax
import jax.numpy as jnp
from jax.experimental import pallas as pl


def kernel(x, edge_index, W, b):
    raise NotImplementedError("write your pallas kernel here")


# speedup vs baseline: 11.6342x; 11.6342x over previous
import functools

import jax
import jax.numpy as jnp
from jax import lax
from jax.experimental import pallas as pl
from jax.experimental.pallas import tpu as pltpu
from jax.experimental.pallas import tpu_sc as plsc

NC = 2
NS = 16
L = 16


def _sc_mesh():
    return plsc.VectorSubcoreMesh(core_axis_name="c", subcore_axis_name="s",
                                  num_cores=NC, num_subcores=NS)


def _sc_degree(dst, n):
    e = dst.shape[0]
    per = e // (NC * NS)
    K = 40
    assert per % K == 0 and per % 8 == 0
    nch = per // K
    DW = 128
    SZ = 640
    ZCH = 80
    last = n - (NS - 1) * SZ
    assert last > 0 and last % ZCH == 0 and SZ % ZCH == 0

    @functools.partial(
        pl.kernel,
        out_type=jax.ShapeDtypeStruct((NC, n, DW), jnp.float32),
        mesh=_sc_mesh(),
        scratch_types=[
            pltpu.VMEM((K, DW), jnp.float32),
            pltpu.VMEM((K,), jnp.int32),
            pltpu.VMEM((ZCH, DW), jnp.float32),
            pltpu.VMEM_SHARED((n, DW), jnp.float32),
        ],
    )
    def deg_kernel(dst_hbm, out_hbm, ones_v, idx_v, zbuf, deg_sh):
        c = lax.axis_index("c")
        s = lax.axis_index("s")
        wid = c * NS + s

        def fill1(i, _):
            for l in range(DW // L):
                ones_v[i, pl.ds(l * L, L)] = jnp.ones((L,), jnp.float32)
            return ()
        lax.fori_loop(0, K, fill1, ())

        def zb(i, _):
            for l in range(DW // L):
                zbuf[i, pl.ds(l * L, L)] = jnp.zeros((L,), jnp.float32)
            return ()
        lax.fori_loop(0, ZCH, zb, ())

        @pl.when(s < NS - 1)
        def _():
            for i in range(SZ // ZCH):
                pltpu.sync_copy(zbuf, deg_sh.at[pl.ds(s * SZ + i * ZCH, ZCH)])

        @pl.when(s == NS - 1)
        def _():
            for i in range(last // ZCH):
                pltpu.sync_copy(
                    zbuf, deg_sh.at[pl.ds((NS - 1) * SZ + i * ZCH, ZCH)])
        plsc.subcore_barrier()

        base = wid * per

        def chunk(j, _):
            pltpu.sync_copy(dst_hbm.at[pl.ds(base + j * K, K)], idx_v)
            pltpu.sync_copy(ones_v, deg_sh.at[idx_v], add=True)
            return ()
        lax.fori_loop(0, nch, chunk, ())
        plsc.subcore_barrier()

        @pl.when(s < NS - 1)
        def _():
            pltpu.sync_copy(deg_sh.at[pl.ds(s * SZ, SZ)],
                            out_hbm.at[c, pl.ds(s * SZ, SZ)])

        @pl.when(s == NS - 1)
        def _():
            pltpu.sync_copy(deg_sh.at[pl.ds((NS - 1) * SZ, last)],
                            out_hbm.at[c, pl.ds((NS - 1) * SZ, last)])

    return deg_kernel(dst)


def _tc_prescale(deg2, x):
    n, d = x.shape
    F = d // 2

    def body(deg_ref, x_ref, o_ref, dinv_ref):
        deg = deg_ref[0, :, 0] + deg_ref[1, :, 0] + 1.0
        dinv = lax.rsqrt(deg)[:, None]
        o_ref[0] = x_ref[:, :F] * dinv
        o_ref[1] = x_ref[:, F:] * dinv
        dinv_ref[...] = dinv

    return pl.pallas_call(
        body,
        out_shape=(jax.ShapeDtypeStruct((2, n, F), jnp.float32),
                   jax.ShapeDtypeStruct((n, 1), jnp.float32)),
    )(deg2, x)


def _sc_aggregate(xs_flat, src, dst, n, F):
    e = src.shape[0]
    per = e // NS
    K = 80
    assert per % K == 0
    nch = per // K
    SZ = 640
    last = n - (NS - 1) * SZ
    assert last > 0 and last % 8 == 0
    NBUF = 2

    @functools.partial(
        pl.kernel,
        out_type=jax.ShapeDtypeStruct((NC, n, F), jnp.float32),
        mesh=_sc_mesh(),
        scratch_types=[
            pltpu.VMEM((NBUF, K), jnp.int32),
            pltpu.VMEM((NBUF, K), jnp.int32),
            pltpu.VMEM((NBUF, K, F), jnp.float32),
            pltpu.VMEM_SHARED((n, F), jnp.float32),
            pltpu.SemaphoreType.DMA((NBUF,)),
        ],
    )
    def agg_kernel(xs_hbm, src_hbm, dst_hbm, out_hbm, gidx, sidx, rows, acc_sh, sems):
        c = lax.axis_index("c")
        s = lax.axis_index("s")

        @pl.when(s < NS - 1)
        def _():
            pltpu.sync_copy(xs_hbm.at[pl.ds(c * n + s * SZ, SZ)],
                            acc_sh.at[pl.ds(s * SZ, SZ)])

        @pl.when(s == NS - 1)
        def _():
            pltpu.sync_copy(xs_hbm.at[pl.ds(c * n + (NS - 1) * SZ, last)],
                            acc_sh.at[pl.ds((NS - 1) * SZ, last)])
        plsc.subcore_barrier()

        base = s * per
        cn = jnp.full((L,), c * n, jnp.int32)

        def load_idx(j, slot):
            pltpu.sync_copy(src_hbm.at[pl.ds(base + j * K, K)],
                            gidx.at[slot])
            pltpu.sync_copy(dst_hbm.at[pl.ds(base + j * K, K)],
                            sidx.at[slot])
            for i in range(K // L):
                gidx[slot, pl.ds(i * L, L)] = gidx[slot, pl.ds(i * L, L)] + cn

        def start_gather(j, slot):
            pltpu.async_copy(xs_hbm.at[gidx.at[slot]], rows.at[slot],
                             sems.at[slot])

        def wait_gather(slot):
            pltpu.make_async_copy(xs_hbm.at[gidx.at[slot]], rows.at[slot],
                                  sems.at[slot]).wait()

        def chunk(j, _):
            load_idx(j, 0)
            start_gather(j, 0)
            wait_gather(0)
            pltpu.sync_copy(rows.at[0], acc_sh.at[sidx.at[0]], add=True)
            return ()
        lax.fori_loop(0, nch, chunk, ())
        plsc.subcore_barrier()

        @pl.when(s < NS - 1)
        def _():
            pltpu.sync_copy(acc_sh.at[pl.ds(s * SZ, SZ)],
                            out_hbm.at[c, pl.ds(s * SZ, SZ)])

        @pl.when(s == NS - 1)
        def _():
            pltpu.sync_copy(acc_sh.at[pl.ds((NS - 1) * SZ, last)],
                            out_hbm.at[c, pl.ds((NS - 1) * SZ, last)])

    return agg_kernel(xs_flat, src, dst)


def _tc_finish(dinv, agg, W, b):
    n = agg.shape[1]
    F = agg.shape[2]
    hid = W.shape[1]
    bn = 2000
    assert n % bn == 0

    def body(dinv_ref, a_ref, w_ref, b_ref, o_ref):
        dv = dinv_ref[...]
        h0 = jnp.dot(a_ref[0] * dv, w_ref[:F, :],
                     preferred_element_type=jnp.float32,
                     precision=lax.Precision.HIGHEST)
        h1 = jnp.dot(a_ref[1] * dv, w_ref[F:, :],
                     preferred_element_type=jnp.float32,
                     precision=lax.Precision.HIGHEST)
        o_ref[...] = jnp.maximum(h0 + h1 + b_ref[0, :], 0.0)

    return pl.pallas_call(
        body,
        grid=(n // bn,),
        in_specs=[
            pl.BlockSpec((bn, 1), lambda i: (i, 0)),
            pl.BlockSpec((2, bn, F), lambda i: (0, i, 0)),
            pl.BlockSpec((2 * F, hid), lambda i: (0, 0)),
            pl.BlockSpec((1, hid), lambda i: (0, 0)),
        ],
        out_specs=pl.BlockSpec((bn, hid), lambda i: (i, 0)),
        out_shape=jax.ShapeDtypeStruct((n, hid), jnp.float32),
    )(dinv, agg, W, b.reshape(1, hid))


def kernel(x, edge_index, W, b):
    n, d = x.shape
    F = d // 2
    src = edge_index[0]
    dst = edge_index[1]
    deg2 = _sc_degree(dst, n)
    xs, dinv = _tc_prescale(deg2, x)
    agg = _sc_aggregate(xs.reshape(2 * n, F), src, dst, n, F)
    return _tc_finish(dinv, agg, W, b)


# speedup vs baseline: 15.8201x; 1.3598x over previous
import functools

import jax
import jax.numpy as jnp
from jax import lax
from jax.experimental import pallas as pl
from jax.experimental.pallas import tpu as pltpu
from jax.experimental.pallas import tpu_sc as plsc

NC = 2
NS = 16
L = 16


def _sc_mesh():
    return plsc.VectorSubcoreMesh(core_axis_name="c", subcore_axis_name="s",
                                  num_cores=NC, num_subcores=NS)


def _sc_degree(dst, n):
    e = dst.shape[0]
    per = e // (NC * NS)
    K = 40
    assert per % K == 0 and per % 8 == 0
    nch = per // K
    DW = 128
    SZ = 640
    ZCH = 80
    last = n - (NS - 1) * SZ
    assert last > 0 and last % ZCH == 0 and SZ % ZCH == 0

    @functools.partial(
        pl.kernel,
        out_type=jax.ShapeDtypeStruct((NC, n, DW), jnp.float32),
        mesh=_sc_mesh(),
        scratch_types=[
            pltpu.VMEM((K, DW), jnp.float32),
            pltpu.VMEM((K,), jnp.int32),
            pltpu.VMEM((ZCH, DW), jnp.float32),
            pltpu.VMEM_SHARED((n, DW), jnp.float32),
        ],
    )
    def deg_kernel(dst_hbm, out_hbm, ones_v, idx_v, zbuf, deg_sh):
        c = lax.axis_index("c")
        s = lax.axis_index("s")
        wid = c * NS + s

        def fill1(i, _):
            for l in range(DW // L):
                ones_v[i, pl.ds(l * L, L)] = jnp.ones((L,), jnp.float32)
            return ()
        lax.fori_loop(0, K, fill1, ())

        def zb(i, _):
            for l in range(DW // L):
                zbuf[i, pl.ds(l * L, L)] = jnp.zeros((L,), jnp.float32)
            return ()
        lax.fori_loop(0, ZCH, zb, ())

        @pl.when(s < NS - 1)
        def _():
            for i in range(SZ // ZCH):
                pltpu.sync_copy(zbuf, deg_sh.at[pl.ds(s * SZ + i * ZCH, ZCH)])

        @pl.when(s == NS - 1)
        def _():
            for i in range(last // ZCH):
                pltpu.sync_copy(
                    zbuf, deg_sh.at[pl.ds((NS - 1) * SZ + i * ZCH, ZCH)])
        plsc.subcore_barrier()

        base = wid * per

        def chunk(j, _):
            pltpu.sync_copy(dst_hbm.at[pl.ds(base + j * K, K)], idx_v)
            pltpu.sync_copy(ones_v, deg_sh.at[idx_v], add=True)
            return ()
        lax.fori_loop(0, nch, chunk, ())
        plsc.subcore_barrier()

        @pl.when(s < NS - 1)
        def _():
            pltpu.sync_copy(deg_sh.at[pl.ds(s * SZ, SZ)],
                            out_hbm.at[c, pl.ds(s * SZ, SZ)])

        @pl.when(s == NS - 1)
        def _():
            pltpu.sync_copy(deg_sh.at[pl.ds((NS - 1) * SZ, last)],
                            out_hbm.at[c, pl.ds((NS - 1) * SZ, last)])

    return deg_kernel(dst)


def _tc_prescale(deg2, x):
    n, d = x.shape
    F = d // 2

    def body(deg_ref, x_ref, o_ref, dinv_ref):
        deg = deg_ref[0, :, 0] + deg_ref[1, :, 0] + 1.0
        dinv = lax.rsqrt(deg)[:, None]
        o_ref[0] = x_ref[:, :F] * dinv
        o_ref[1] = x_ref[:, F:] * dinv
        dinv_ref[...] = dinv

    return pl.pallas_call(
        body,
        out_shape=(jax.ShapeDtypeStruct((2, n, F), jnp.float32),
                   jax.ShapeDtypeStruct((n, 1), jnp.float32)),
    )(deg2, x)


def _sc_aggregate(xs_flat, src, dst, n, F):
    e = src.shape[0]
    per = e // NS
    K = 80
    assert per % K == 0
    nch = per // K
    SZ = 640
    last = n - (NS - 1) * SZ
    assert last > 0 and last % 8 == 0
    NBUF = 4

    @functools.partial(
        pl.kernel,
        out_type=jax.ShapeDtypeStruct((NC, n, F), jnp.float32),
        mesh=_sc_mesh(),
        scratch_types=[
            pltpu.VMEM((NBUF, K), jnp.int32),
            pltpu.VMEM((NBUF, K), jnp.int32),
            pltpu.VMEM((NBUF, K, F), jnp.float32),
            pltpu.VMEM_SHARED((n, F), jnp.float32),
            pltpu.SemaphoreType.DMA((NBUF,)),
        ],
    )
    def agg_kernel(xs_hbm, src_hbm, dst_hbm, out_hbm, gidx, sidx, rows, acc_sh, sems):
        c = lax.axis_index("c")
        s = lax.axis_index("s")

        @pl.when(s < NS - 1)
        def _():
            pltpu.sync_copy(xs_hbm.at[pl.ds(c * n + s * SZ, SZ)],
                            acc_sh.at[pl.ds(s * SZ, SZ)])

        @pl.when(s == NS - 1)
        def _():
            pltpu.sync_copy(xs_hbm.at[pl.ds(c * n + (NS - 1) * SZ, last)],
                            acc_sh.at[pl.ds((NS - 1) * SZ, last)])
        plsc.subcore_barrier()

        base = s * per
        cn = jnp.full((L,), c * n, jnp.int32)

        def load_idx(j, slot):
            pltpu.sync_copy(src_hbm.at[pl.ds(base + j * K, K)],
                            gidx.at[slot])
            pltpu.sync_copy(dst_hbm.at[pl.ds(base + j * K, K)],
                            sidx.at[slot])
            for i in range(K // L):
                gidx[slot, pl.ds(i * L, L)] = gidx[slot, pl.ds(i * L, L)] + cn

        def start_gather(j, slot):
            pltpu.async_copy(xs_hbm.at[gidx.at[slot]], rows.at[slot],
                             sems.at[slot])

        def wait_gather(slot):
            pltpu.make_async_copy(xs_hbm.at[gidx.at[slot]], rows.at[slot],
                                  sems.at[slot]).wait()

        for b in range(NBUF - 1):
            load_idx(b, b)
            start_gather(b, b)

        def chunk(j, _):
            slot = lax.rem(j, NBUF)
            nslot = lax.rem(j + NBUF - 1, NBUF)

            @pl.when(j + NBUF - 1 < nch)
            def _():
                load_idx(j + NBUF - 1, nslot)
                start_gather(j + NBUF - 1, nslot)

            wait_gather(slot)
            pltpu.sync_copy(rows.at[slot], acc_sh.at[sidx.at[slot]], add=True)
            return ()
        lax.fori_loop(0, nch, chunk, ())
        plsc.subcore_barrier()

        @pl.when(s < NS - 1)
        def _():
            pltpu.sync_copy(acc_sh.at[pl.ds(s * SZ, SZ)],
                            out_hbm.at[c, pl.ds(s * SZ, SZ)])

        @pl.when(s == NS - 1)
        def _():
            pltpu.sync_copy(acc_sh.at[pl.ds((NS - 1) * SZ, last)],
                            out_hbm.at[c, pl.ds((NS - 1) * SZ, last)])

    return agg_kernel(xs_flat, src, dst)


def _tc_finish(dinv, agg, W, b):
    n = agg.shape[1]
    F = agg.shape[2]
    hid = W.shape[1]
    bn = 2000
    assert n % bn == 0

    def body(dinv_ref, a_ref, w_ref, b_ref, o_ref):
        dv = dinv_ref[...]
        h0 = jnp.dot(a_ref[0] * dv, w_ref[:F, :],
                     preferred_element_type=jnp.float32,
                     precision=lax.Precision.HIGHEST)
        h1 = jnp.dot(a_ref[1] * dv, w_ref[F:, :],
                     preferred_element_type=jnp.float32,
                     precision=lax.Precision.HIGHEST)
        o_ref[...] = jnp.maximum(h0 + h1 + b_ref[0, :], 0.0)

    return pl.pallas_call(
        body,
        grid=(n // bn,),
        in_specs=[
            pl.BlockSpec((bn, 1), lambda i: (i, 0)),
            pl.BlockSpec((2, bn, F), lambda i: (0, i, 0)),
            pl.BlockSpec((2 * F, hid), lambda i: (0, 0)),
            pl.BlockSpec((1, hid), lambda i: (0, 0)),
        ],
        out_specs=pl.BlockSpec((bn, hid), lambda i: (i, 0)),
        out_shape=jax.ShapeDtypeStruct((n, hid), jnp.float32),
    )(dinv, agg, W, b.reshape(1, hid))


def kernel(x, edge_index, W, b):
    n, d = x.shape
    F = d // 2
    src = edge_index[0]
    dst = edge_index[1]
    deg2 = _sc_degree(dst, n)
    xs, dinv = _tc_prescale(deg2, x)
    agg = _sc_aggregate(xs.reshape(2 * n, F), src, dst, n, F)
    return _tc_finish(dinv, agg, W, b)


# speedup vs baseline: 23.4958x; 1.4852x over previous
import functools

import jax
import jax.numpy as jnp
from jax import lax
from jax.experimental import pallas as pl
from jax.experimental.pallas import tpu as pltpu
from jax.experimental.pallas import tpu_sc as plsc

NC = 2
NS = 16
L = 16


def _sc_mesh():
    return plsc.VectorSubcoreMesh(core_axis_name="c", subcore_axis_name="s",
                                  num_cores=NC, num_subcores=NS)


def _sc_degree(dst, n):
    e = dst.shape[0]
    per = e // (NC * NS)
    K = 40
    assert per % K == 0 and per % 8 == 0
    nch = per // K
    DW = 128
    SZ = 640
    ZCH = 80
    last = n - (NS - 1) * SZ
    assert last > 0 and last % ZCH == 0 and SZ % ZCH == 0

    @functools.partial(
        pl.kernel,
        out_type=jax.ShapeDtypeStruct((NC, n, DW), jnp.float32),
        mesh=_sc_mesh(),
        scratch_types=[
            pltpu.VMEM((K, DW), jnp.float32),
            pltpu.VMEM((3, K), jnp.int32),
            pltpu.VMEM((ZCH, DW), jnp.float32),
            pltpu.VMEM_SHARED((n, DW), jnp.float32),
            pltpu.SemaphoreType.DMA((3,)),
        ],
    )
    def deg_kernel(dst_hbm, out_hbm, ones_v, idx_v, zbuf, deg_sh, isems):
        c = lax.axis_index("c")
        s = lax.axis_index("s")
        wid = c * NS + s

        def fill1(i, _):
            for l in range(DW // L):
                ones_v[i, pl.ds(l * L, L)] = jnp.ones((L,), jnp.float32)
            return ()
        lax.fori_loop(0, K, fill1, ())

        def zb(i, _):
            for l in range(DW // L):
                zbuf[i, pl.ds(l * L, L)] = jnp.zeros((L,), jnp.float32)
            return ()
        lax.fori_loop(0, ZCH, zb, ())

        @pl.when(s < NS - 1)
        def _():
            for i in range(SZ // ZCH):
                pltpu.sync_copy(zbuf, deg_sh.at[pl.ds(s * SZ + i * ZCH, ZCH)])

        @pl.when(s == NS - 1)
        def _():
            for i in range(last // ZCH):
                pltpu.sync_copy(
                    zbuf, deg_sh.at[pl.ds((NS - 1) * SZ + i * ZCH, ZCH)])
        plsc.subcore_barrier()

        base = wid * per

        def load_idx_async(j, slot):
            pltpu.async_copy(dst_hbm.at[pl.ds(base + j * K, K)],
                             idx_v.at[slot], isems.at[slot])

        def wait_idx(j, slot):
            pltpu.make_async_copy(dst_hbm.at[pl.ds(base + j * K, K)],
                                  idx_v.at[slot], isems.at[slot]).wait()

        load_idx_async(0, 0)
        load_idx_async(1, 1)

        def chunk(j, _):
            s0 = lax.rem(j, 3)
            s2 = lax.rem(j + 2, 3)

            @pl.when(j + 2 < nch)
            def _():
                load_idx_async(j + 2, s2)

            wait_idx(j, s0)
            pltpu.sync_copy(ones_v, deg_sh.at[idx_v.at[s0]], add=True)
            return ()
        lax.fori_loop(0, nch, chunk, ())
        plsc.subcore_barrier()

        @pl.when(s < NS - 1)
        def _():
            pltpu.sync_copy(deg_sh.at[pl.ds(s * SZ, SZ)],
                            out_hbm.at[c, pl.ds(s * SZ, SZ)])

        @pl.when(s == NS - 1)
        def _():
            pltpu.sync_copy(deg_sh.at[pl.ds((NS - 1) * SZ, last)],
                            out_hbm.at[c, pl.ds((NS - 1) * SZ, last)])

    return deg_kernel(dst)


def _tc_prescale(deg2, x):
    n, d = x.shape
    F = d // 2

    def body(deg_ref, x_ref, o_ref, dinv_ref):
        deg = deg_ref[0, :, 0] + deg_ref[1, :, 0] + 1.0
        dinv = lax.rsqrt(deg)[:, None]
        o_ref[0] = x_ref[:, :F] * dinv
        o_ref[1] = x_ref[:, F:] * dinv
        dinv_ref[...] = dinv

    return pl.pallas_call(
        body,
        out_shape=(jax.ShapeDtypeStruct((2, n, F), jnp.float32),
                   jax.ShapeDtypeStruct((n, 1), jnp.float32)),
    )(deg2, x)


def _sc_aggregate(xs_flat, src, dst, n, F):
    e = src.shape[0]
    per = e // NS
    K = 80
    assert per % K == 0
    nch = per // K
    SZ = 640
    last = n - (NS - 1) * SZ
    assert last > 0 and last % 8 == 0
    NBUF = 4

    @functools.partial(
        pl.kernel,
        out_type=jax.ShapeDtypeStruct((NC, n, F), jnp.float32),
        mesh=_sc_mesh(),
        scratch_types=[
            pltpu.VMEM((NBUF, K), jnp.int32),
            pltpu.VMEM((NBUF, K), jnp.int32),
            pltpu.VMEM((NBUF, K, F), jnp.float32),
            pltpu.VMEM_SHARED((n, F), jnp.float32),
            pltpu.SemaphoreType.DMA((NBUF,)),
            pltpu.SemaphoreType.DMA((NBUF,)),
        ],
    )
    def agg_kernel(xs_hbm, src_hbm, dst_hbm, out_hbm, gidx, sidx, rows, acc_sh,
                   sems, isems):
        c = lax.axis_index("c")
        s = lax.axis_index("s")

        @pl.when(s < NS - 1)
        def _():
            pltpu.sync_copy(xs_hbm.at[pl.ds(c * n + s * SZ, SZ)],
                            acc_sh.at[pl.ds(s * SZ, SZ)])

        @pl.when(s == NS - 1)
        def _():
            pltpu.sync_copy(xs_hbm.at[pl.ds(c * n + (NS - 1) * SZ, last)],
                            acc_sh.at[pl.ds((NS - 1) * SZ, last)])
        plsc.subcore_barrier()

        base = s * per
        cn = jnp.full((L,), c * n, jnp.int32)

        def load_idx_async(j, slot):
            pltpu.async_copy(src_hbm.at[pl.ds(base + j * K, K)],
                             gidx.at[slot], isems.at[slot])
            pltpu.async_copy(dst_hbm.at[pl.ds(base + j * K, K)],
                             sidx.at[slot], isems.at[slot])

        def wait_idx(j, slot):
            pltpu.make_async_copy(src_hbm.at[pl.ds(base + j * K, K)],
                                  gidx.at[slot], isems.at[slot]).wait()
            pltpu.make_async_copy(dst_hbm.at[pl.ds(base + j * K, K)],
                                  sidx.at[slot], isems.at[slot]).wait()

        def add_cn(slot):
            for i in range(K // L):
                gidx[slot, pl.ds(i * L, L)] = gidx[slot, pl.ds(i * L, L)] + cn

        def start_gather(j, slot):
            pltpu.async_copy(xs_hbm.at[gidx.at[slot]], rows.at[slot],
                             sems.at[slot])

        def wait_gather(slot):
            pltpu.make_async_copy(xs_hbm.at[gidx.at[slot]], rows.at[slot],
                                  sems.at[slot]).wait()

        load_idx_async(0, 0)
        load_idx_async(1, 1)
        wait_idx(0, 0)
        add_cn(0)
        start_gather(0, 0)

        def chunk(j, _):
            s0 = lax.rem(j, NBUF)
            s1 = lax.rem(j + 1, NBUF)
            s2 = lax.rem(j + 2, NBUF)

            @pl.when(j + 2 < nch)
            def _():
                load_idx_async(j + 2, s2)

            @pl.when(j + 1 < nch)
            def _():
                wait_idx(j + 1, s1)
                add_cn(s1)
                start_gather(j + 1, s1)

            wait_gather(s0)
            pltpu.sync_copy(rows.at[s0], acc_sh.at[sidx.at[s0]], add=True)
            return ()
        lax.fori_loop(0, nch, chunk, ())
        plsc.subcore_barrier()

        @pl.when(s < NS - 1)
        def _():
            pltpu.sync_copy(acc_sh.at[pl.ds(s * SZ, SZ)],
                            out_hbm.at[c, pl.ds(s * SZ, SZ)])

        @pl.when(s == NS - 1)
        def _():
            pltpu.sync_copy(acc_sh.at[pl.ds((NS - 1) * SZ, last)],
                            out_hbm.at[c, pl.ds((NS - 1) * SZ, last)])

    return agg_kernel(xs_flat, src, dst)


def _tc_finish(dinv, agg, W, b):
    n = agg.shape[1]
    F = agg.shape[2]
    hid = W.shape[1]
    bn = 2000
    assert n % bn == 0

    def body(dinv_ref, a_ref, w_ref, b_ref, o_ref):
        dv = dinv_ref[...]
        h0 = jnp.dot(a_ref[0] * dv, w_ref[:F, :],
                     preferred_element_type=jnp.float32,
                     precision=lax.Precision.HIGHEST)
        h1 = jnp.dot(a_ref[1] * dv, w_ref[F:, :],
                     preferred_element_type=jnp.float32,
                     precision=lax.Precision.HIGHEST)
        o_ref[...] = jnp.maximum(h0 + h1 + b_ref[0, :], 0.0)

    return pl.pallas_call(
        body,
        grid=(n // bn,),
        in_specs=[
            pl.BlockSpec((bn, 1), lambda i: (i, 0)),
            pl.BlockSpec((2, bn, F), lambda i: (0, i, 0)),
            pl.BlockSpec((2 * F, hid), lambda i: (0, 0)),
            pl.BlockSpec((1, hid), lambda i: (0, 0)),
        ],
        out_specs=pl.BlockSpec((bn, hid), lambda i: (i, 0)),
        out_shape=jax.ShapeDtypeStruct((n, hid), jnp.float32),
    )(dinv, agg, W, b.reshape(1, hid))


def kernel(x, edge_index, W, b):
    n, d = x.shape
    F = d // 2
    src = edge_index[0]
    dst = edge_index[1]
    deg2 = _sc_degree(dst, n)
    xs, dinv = _tc_prescale(deg2, x)
    agg = _sc_aggregate(xs.reshape(2 * n, F), src, dst, n, F)
    return _tc_finish(dinv, agg, W, b)


# speedup vs baseline: 26.1621x; 1.1135x over previous
import functools

import jax
import jax.numpy as jnp
from jax import lax
from jax.experimental import pallas as pl
from jax.experimental.pallas import tpu as pltpu
from jax.experimental.pallas import tpu_sc as plsc

NC = 2
NS = 16
L = 16


def _sc_mesh():
    return plsc.VectorSubcoreMesh(core_axis_name="c", subcore_axis_name="s",
                                  num_cores=NC, num_subcores=NS)


def _sc_degree(dst, n):
    e = dst.shape[0]
    per = e // (NC * NS)
    K = 40
    assert per % K == 0 and per % 8 == 0
    nch = per // K
    DW = 128
    SZ = 640
    ZCH = 80
    last = n - (NS - 1) * SZ
    assert last > 0 and last % ZCH == 0 and SZ % ZCH == 0

    @functools.partial(
        pl.kernel,
        out_type=jax.ShapeDtypeStruct((NC, n, DW), jnp.float32),
        mesh=_sc_mesh(),
        scratch_types=[
            pltpu.VMEM((K, DW), jnp.float32),
            pltpu.VMEM((3, K), jnp.int32),
            pltpu.VMEM((ZCH, DW), jnp.float32),
            pltpu.VMEM_SHARED((n, DW), jnp.float32),
            pltpu.SemaphoreType.DMA((3,)),
        ],
    )
    def deg_kernel(dst_hbm, out_hbm, ones_v, idx_v, zbuf, deg_sh, isems):
        c = lax.axis_index("c")
        s = lax.axis_index("s")
        wid = c * NS + s

        def fill1(i, _):
            for l in range(DW // L):
                ones_v[i, pl.ds(l * L, L)] = jnp.ones((L,), jnp.float32)
            return ()
        lax.fori_loop(0, K, fill1, ())

        def zb(i, _):
            for l in range(DW // L):
                zbuf[i, pl.ds(l * L, L)] = jnp.zeros((L,), jnp.float32)
            return ()
        lax.fori_loop(0, ZCH, zb, ())

        @pl.when(s < NS - 1)
        def _():
            for i in range(SZ // ZCH):
                pltpu.sync_copy(zbuf, deg_sh.at[pl.ds(s * SZ + i * ZCH, ZCH)])

        @pl.when(s == NS - 1)
        def _():
            for i in range(last // ZCH):
                pltpu.sync_copy(
                    zbuf, deg_sh.at[pl.ds((NS - 1) * SZ + i * ZCH, ZCH)])
        plsc.subcore_barrier()

        base = wid * per

        def load_idx_async(j, slot):
            pltpu.async_copy(dst_hbm.at[pl.ds(base + j * K, K)],
                             idx_v.at[slot], isems.at[slot])

        def wait_idx(j, slot):
            pltpu.make_async_copy(dst_hbm.at[pl.ds(base + j * K, K)],
                                  idx_v.at[slot], isems.at[slot]).wait()

        load_idx_async(0, 0)
        load_idx_async(1, 1)

        def chunk(j, _):
            s0 = lax.rem(j, 3)
            s2 = lax.rem(j + 2, 3)

            @pl.when(j + 2 < nch)
            def _():
                load_idx_async(j + 2, s2)

            wait_idx(j, s0)
            pltpu.sync_copy(ones_v, deg_sh.at[idx_v.at[s0]], add=True)
            return ()
        lax.fori_loop(0, nch, chunk, ())
        plsc.subcore_barrier()

        @pl.when(s < NS - 1)
        def _():
            pltpu.sync_copy(deg_sh.at[pl.ds(s * SZ, SZ)],
                            out_hbm.at[c, pl.ds(s * SZ, SZ)])

        @pl.when(s == NS - 1)
        def _():
            pltpu.sync_copy(deg_sh.at[pl.ds((NS - 1) * SZ, last)],
                            out_hbm.at[c, pl.ds((NS - 1) * SZ, last)])

    return deg_kernel(dst)


def _tc_prescale(deg2, x):
    n, d = x.shape
    F = d // 2

    bn = 2000
    assert n % bn == 0

    def body(deg_ref, x_ref, o_ref, dinv_ref):
        deg = deg_ref[0, :, 0] + deg_ref[1, :, 0] + 1.0
        dinv = lax.rsqrt(deg)[:, None]
        o_ref[0] = x_ref[:, :F] * dinv
        o_ref[1] = x_ref[:, F:] * dinv
        dinv_ref[...] = dinv

    return pl.pallas_call(
        body,
        grid=(n // bn,),
        in_specs=[
            pl.BlockSpec((2, bn, 128), lambda i: (0, i, 0)),
            pl.BlockSpec((bn, d), lambda i: (i, 0)),
        ],
        out_specs=(pl.BlockSpec((2, bn, F), lambda i: (0, i, 0)),
                   pl.BlockSpec((bn, 1), lambda i: (i, 0))),
        out_shape=(jax.ShapeDtypeStruct((2, n, F), jnp.float32),
                   jax.ShapeDtypeStruct((n, 1), jnp.float32)),
    )(deg2, x)


def _sc_aggregate(xs_flat, src, dst, n, F):
    e = src.shape[0]
    per = e // NS
    K = 80
    assert per % K == 0
    nch = per // K
    SZ = 640
    last = n - (NS - 1) * SZ
    assert last > 0 and last % 8 == 0
    NBUF = 4

    @functools.partial(
        pl.kernel,
        out_type=jax.ShapeDtypeStruct((NC, n, F), jnp.float32),
        mesh=_sc_mesh(),
        scratch_types=[
            pltpu.VMEM((NBUF, K), jnp.int32),
            pltpu.VMEM((NBUF, K), jnp.int32),
            pltpu.VMEM((NBUF, K, F), jnp.float32),
            pltpu.VMEM_SHARED((n, F), jnp.float32),
            pltpu.SemaphoreType.DMA((NBUF,)),
            pltpu.SemaphoreType.DMA((NBUF,)),
        ],
    )
    def agg_kernel(xs_hbm, src_hbm, dst_hbm, out_hbm, gidx, sidx, rows, acc_sh,
                   sems, isems):
        c = lax.axis_index("c")
        s = lax.axis_index("s")

        @pl.when(s < NS - 1)
        def _():
            pltpu.sync_copy(xs_hbm.at[pl.ds(c * n + s * SZ, SZ)],
                            acc_sh.at[pl.ds(s * SZ, SZ)])

        @pl.when(s == NS - 1)
        def _():
            pltpu.sync_copy(xs_hbm.at[pl.ds(c * n + (NS - 1) * SZ, last)],
                            acc_sh.at[pl.ds((NS - 1) * SZ, last)])
        plsc.subcore_barrier()

        base = s * per
        cn = jnp.full((L,), c * n, jnp.int32)

        def load_idx_async(j, slot):
            pltpu.async_copy(src_hbm.at[pl.ds(base + j * K, K)],
                             gidx.at[slot], isems.at[slot])
            pltpu.async_copy(dst_hbm.at[pl.ds(base + j * K, K)],
                             sidx.at[slot], isems.at[slot])

        def wait_idx(j, slot):
            pltpu.make_async_copy(src_hbm.at[pl.ds(base + j * K, K)],
                                  gidx.at[slot], isems.at[slot]).wait()
            pltpu.make_async_copy(dst_hbm.at[pl.ds(base + j * K, K)],
                                  sidx.at[slot], isems.at[slot]).wait()

        def add_cn(slot):
            for i in range(K // L):
                gidx[slot, pl.ds(i * L, L)] = gidx[slot, pl.ds(i * L, L)] + cn

        def start_gather(j, slot):
            pltpu.async_copy(xs_hbm.at[gidx.at[slot]], rows.at[slot],
                             sems.at[slot])

        def wait_gather(slot):
            pltpu.make_async_copy(xs_hbm.at[gidx.at[slot]], rows.at[slot],
                                  sems.at[slot]).wait()

        load_idx_async(0, 0)
        load_idx_async(1, 1)
        load_idx_async(2, 2)
        wait_idx(0, 0)
        add_cn(0)
        start_gather(0, 0)
        wait_idx(1, 1)
        add_cn(1)
        start_gather(1, 1)

        def chunk(j, _):
            s0 = lax.rem(j, NBUF)
            s2 = lax.rem(j + 2, NBUF)
            s3 = lax.rem(j + 3, NBUF)

            @pl.when(j + 3 < nch)
            def _():
                load_idx_async(j + 3, s3)

            @pl.when(j + 2 < nch)
            def _():
                wait_idx(j + 2, s2)
                add_cn(s2)
                start_gather(j + 2, s2)

            wait_gather(s0)
            pltpu.sync_copy(rows.at[s0], acc_sh.at[sidx.at[s0]], add=True)
            return ()
        lax.fori_loop(0, nch, chunk, ())
        plsc.subcore_barrier()

        @pl.when(s < NS - 1)
        def _():
            pltpu.sync_copy(acc_sh.at[pl.ds(s * SZ, SZ)],
                            out_hbm.at[c, pl.ds(s * SZ, SZ)])

        @pl.when(s == NS - 1)
        def _():
            pltpu.sync_copy(acc_sh.at[pl.ds((NS - 1) * SZ, last)],
                            out_hbm.at[c, pl.ds((NS - 1) * SZ, last)])

    return agg_kernel(xs_flat, src, dst)


def _tc_finish(dinv, agg, W, b):
    n = agg.shape[1]
    F = agg.shape[2]
    hid = W.shape[1]
    bn = 1000
    assert n % bn == 0

    def body(dinv_ref, a_ref, w_ref, b_ref, o_ref):
        dv = dinv_ref[...]
        h0 = jnp.dot(a_ref[0] * dv, w_ref[:F, :],
                     preferred_element_type=jnp.float32,
                     precision=lax.Precision.HIGHEST)
        h1 = jnp.dot(a_ref[1] * dv, w_ref[F:, :],
                     preferred_element_type=jnp.float32,
                     precision=lax.Precision.HIGHEST)
        o_ref[...] = jnp.maximum(h0 + h1 + b_ref[0, :], 0.0)

    return pl.pallas_call(
        body,
        grid=(n // bn,),
        in_specs=[
            pl.BlockSpec((bn, 1), lambda i: (i, 0)),
            pl.BlockSpec((2, bn, F), lambda i: (0, i, 0)),
            pl.BlockSpec((2 * F, hid), lambda i: (0, 0)),
            pl.BlockSpec((1, hid), lambda i: (0, 0)),
        ],
        out_specs=pl.BlockSpec((bn, hid), lambda i: (i, 0)),
        out_shape=jax.ShapeDtypeStruct((n, hid), jnp.float32),
    )(dinv, agg, W, b.reshape(1, hid))


def kernel(x, edge_index, W, b):
    n, d = x.shape
    F = d // 2
    src = edge_index[0]
    dst = edge_index[1]
    deg2 = _sc_degree(dst, n)
    xs, dinv = _tc_prescale(deg2, x)
    agg = _sc_aggregate(xs.reshape(2 * n, F), src, dst, n, F)
    return _tc_finish(dinv, agg, W, b)
